# Initial kernel scaffold; baseline (speedup 1.0000x reference)
import jax
import jax.numpy as jnp
from jax.experimental import pallas as pl


def kernel(batch_idxs, user_embed, product_embed, relation_vec, relation_bias):
    raise NotImplementedError("write your pallas kernel here")


# speedup vs baseline: 2.5621x; 2.5621x over previous
import functools

import jax
import jax.numpy as jnp
from jax import lax
from jax.experimental import pallas as pl
from jax.experimental.pallas import tpu as pltpu, tpu_sc as plsc

EMBED = 64
NEG = 5
COLS = 2 + NEG
B = 16384
CHUNK = 64
PCHUNK = CHUNK * (1 + NEG)
TBLK = 2048


def _sc_logits_kernel(nw):
    rows_per_w = B // nw
    nchunk = rows_per_w // CHUNK
    mesh = plsc.VectorSubcoreMesh(core_axis_name="c", subcore_axis_name="s")
    nc = mesh.num_cores

    scratch = [
        pltpu.VMEM((CHUNK, COLS), jnp.int32),
        [pltpu.VMEM((CHUNK,), jnp.int32)] * 2,
        [pltpu.VMEM((PCHUNK,), jnp.int32)] * 2,
        [pltpu.VMEM((PCHUNK,), jnp.float32)] * 2,
        [pltpu.VMEM((CHUNK, EMBED), jnp.float32)] * 2,
        [pltpu.VMEM((PCHUNK, EMBED), jnp.float32)] * 2,
        pltpu.VMEM((CHUNK, 16), jnp.float32),
        pltpu.VMEM((EMBED,), jnp.float32),
        [pltpu.SemaphoreType.DMA] * 2,
    ]

    @functools.partial(
        pl.kernel,
        out_type=jax.ShapeDtypeStruct((B, 16), jnp.float32),
        mesh=mesh,
        scratch_types=scratch,
        compiler_params=pltpu.CompilerParams(
            needs_layout_passes=False, use_tc_tiling_on_sc=False),
    )
    def body(bidx_hbm, user_hbm, prod_hbm, rel_hbm, bias_hbm, out_hbm,
             idx2d, hidx, pidx, biasv, headv, prodv, logitsv, relv, sems):
        wid = lax.axis_index("s") * nc + lax.axis_index("c")
        wstart = wid * rows_per_w
        lane = jnp.arange(16, dtype=jnp.int32)

        pltpu.sync_copy(rel_hbm, relv)
        rel = [relv[pl.ds(16 * k, 16)] for k in range(EMBED // 16)]

        pend = {}

        def stage(c):
            bb = c % 2
            base_row = wstart + c * CHUNK
            pltpu.sync_copy(bidx_hbm.at[pl.ds(base_row, CHUNK), :], idx2d)
            zero = jnp.zeros((16,), jnp.int32)
            for g in range(CHUNK // 16):
                r = g * 16 + lane
                hidx[bb][pl.ds(g * 16, 16)] = plsc.load_gather(idx2d, [r, zero])
            for g in range(PCHUNK // 16):
                p = g * 16 + lane
                r = lax.div(p, jnp.int32(1 + NEG))
                cc = p - r * (1 + NEG) + 1
                pidx[bb][pl.ds(g * 16, 16)] = plsc.load_gather(idx2d, [r, cc])
            pend[c] = (
                pltpu.async_copy(bias_hbm.at[pidx[bb]], biasv[bb], sems[bb]),
                pltpu.async_copy(user_hbm.at[hidx[bb]], headv[bb], sems[bb]),
                pltpu.async_copy(prod_hbm.at[pidx[bb]], prodv[bb], sems[bb]),
            )

        def compute(c):
            bb = c % 2
            for d in pend.pop(c):
                d.wait()
            hv, pv, bv = headv[bb], prodv[bb], biasv[bb]

            def row(r, carry):
                ex = [hv[r, pl.ds(16 * k, 16)] + rel[k]
                      for k in range(EMBED // 16)]
                base6 = r * (1 + NEG)
                acc = jnp.zeros((16,), jnp.float32)
                for j in range(1 + NEG):
                    q = [pv[base6 + j, pl.ds(16 * k, 16)]
                         for k in range(EMBED // 16)]
                    part = ex[0] * q[0] + ex[1] * q[1] + ex[2] * q[2] + ex[3] * q[3]
                    acc = jnp.where(lane == j, jnp.sum(part), acc)
                bsel = lane < (1 + NEG)
                bidx16 = jnp.where(bsel, base6 + lane, 0)
                bias16 = jnp.where(bsel, plsc.load_gather(bv, [bidx16]), 0.0)
                logitsv[r, :] = acc + bias16
                return carry

            lax.fori_loop(0, CHUNK, row, 0)
            base_row = wstart + c * CHUNK
            pltpu.sync_copy(logitsv, out_hbm.at[pl.ds(base_row, CHUNK), :])

        stage(0)
        for c in range(nchunk):
            if c + 1 < nchunk:
                stage(c + 1)
            compute(c)

    return body


def _tc_loss_body(x_ref, o_ref):
    x = x_ref[...]
    lane = lax.broadcasted_iota(jnp.int32, x.shape, 1)
    ls_pos = jax.nn.log_sigmoid(x)
    ls_neg = jax.nn.log_sigmoid(-x)
    pos_term = jnp.sum(jnp.where(lane == 0, ls_pos, 0.0), axis=-1)
    neg_term = jnp.sum(jnp.where((lane >= 1) & (lane < 1 + NEG), ls_neg, 0.0),
                       axis=-1)
    o_ref[...] = -pos_term - neg_term


@jax.jit
def kernel(batch_idxs, user_embed, product_embed, relation_vec, relation_bias):
    info = plsc.get_sparse_core_info()
    nw = info.num_cores * info.num_subcores
    bidx = batch_idxs.astype(jnp.int32)
    rel = relation_vec.reshape((EMBED,)).astype(jnp.float32)

    logits = _sc_logits_kernel(nw)(
        bidx, user_embed, product_embed, rel, relation_bias)

    loss = pl.pallas_call(
        _tc_loss_body,
        out_shape=jax.ShapeDtypeStruct((B,), jnp.float32),
        grid=(B // TBLK,),
        in_specs=[pl.BlockSpec((TBLK, 16), lambda i: (i, 0))],
        out_specs=pl.BlockSpec((TBLK,), lambda i: (i,)),
    )(logits)
    return loss
